# Initial kernel scaffold; baseline (speedup 1.0000x reference)
import jax
import jax.numpy as jnp
from jax.experimental import pallas as pl


def kernel(x, C_mat, C_vec):
    raise NotImplementedError("write your pallas kernel here")


# speedup vs baseline: 8.8824x; 8.8824x over previous
import numpy as np

import jax
import jax.numpy as jnp
from jax import lax
from jax.experimental import pallas as pl
from jax.experimental.pallas import tpu as pltpu
from jax.experimental.pallas import tpu_sc as plsc

N_PTS = 262144
RES = 512
ODIM = 32
NC, NS = 2, 16
NW = NC * NS
PTS_PER_W = N_PTS // NW
P = 64
N_CHUNKS = PTS_PER_W // P

_MAT_IDS = ((0, 1), (0, 2), (1, 2))
_VEC_IDS = (2, 1, 0)
_LANES = np.arange(16, dtype=np.int32)


def _prep_coord(c):
    p = c * (RES / 2.0) + (RES / 2.0 - 0.5)
    fi = p.astype(jnp.int32)
    fi = jnp.where(fi.astype(jnp.float32) > p, fi - 1, fi)
    t = p - fi.astype(jnp.float32)
    i0 = jnp.clip(fi, 0, RES - 1)
    i1 = jnp.clip(fi + 1, 0, RES - 1)
    v0 = ((fi >= 0) & (fi <= RES - 1)).astype(jnp.float32)
    v1 = ((fi >= -1) & (fi <= RES - 2)).astype(jnp.float32)
    w0 = (1.0 - t) * v0
    w1 = t * v1
    return i0, i1, w0, w1


_GDN = lax.GatherDimensionNumbers(
    offset_dims=(), collapsed_slice_dims=(0,), start_index_map=(0,))


def _bcast(vec, slot, zeros):
    idx = (zeros + slot).reshape(16, 1)
    return lax.gather(vec, idx, dimension_numbers=_GDN, slice_sizes=(1,),
                      mode=lax.GatherScatterMode.PROMISE_IN_BOUNDS)


def _body(xT_hbm, mat_hbm, vec_hbm, out_hbm,
          xcols, widx, wall, rows, outbuf, sem):
    wid = lax.axis_index("s") * NC + lax.axis_index("c")
    lanes = lax.iota(jnp.int32, 16)
    zeros = lanes * 0

    def chunk_body(c, _):
        base = pl.multiple_of(wid * PTS_PER_W + c * P, P)

        for j in range(3):
            pltpu.sync_copy(xT_hbm.at[j, pl.ds(base, P)], xcols.at[j])

        for g in range(P // 16):
            sl = pl.ds(g * 16, 16)
            prow = g * 16 + lanes
            pre = [_prep_coord(xcols[j, sl]) for j in range(3)]
            for i in range(3):
                a, b = _MAT_IDS[i]
                xi0, xi1, wx0, wx1 = pre[a]
                yi0, yi1, wy0, wy1 = pre[b]
                pbase = i * (RES * RES)
                r0 = pbase + yi0 * RES
                r1 = pbase + yi1 * RES
                widx[4 * i + 0, sl] = r0 + xi0
                widx[4 * i + 1, sl] = r0 + xi1
                widx[4 * i + 2, sl] = r1 + xi0
                widx[4 * i + 3, sl] = r1 + xi1
                zi0, zi1, wz0, wz1 = pre[_VEC_IDS[i]]
                widx[12 + 2 * i + 0, sl] = i * RES + zi0
                widx[12 + 2 * i + 1, sl] = i * RES + zi1
                wmats = (wy0 * wx0, wy0 * wx1, wy1 * wx0, wy1 * wx1)
                for k in range(4):
                    plsc.store_scatter(
                        wall, [prow, zeros + (4 * i + k)], wmats[k])
                plsc.store_scatter(wall, [prow, zeros + (16 + 2 * i)], wz0)
                plsc.store_scatter(wall, [prow, zeros + (17 + 2 * i)], wz1)

        descs = [pltpu.async_copy(mat_hbm.at[widx.at[j]], rows.at[j], sem)
                 for j in range(12)]
        descs += [pltpu.async_copy(vec_hbm.at[widx.at[12 + j]],
                                   rows.at[12 + j], sem)
                  for j in range(6)]
        for d in descs:
            d.wait()

        def point_body(p_, carry):
            wlo = wall[p_, pl.ds(0, 16)]
            whi = wall[p_, pl.ds(16, 16)]
            for i in range(3):
                w = [_bcast(wlo, 4 * i + k, zeros) for k in range(4)]
                u0 = _bcast(whi, 2 * i + 0, zeros)
                u1 = _bcast(whi, 2 * i + 1, zeros)
                for h in range(2):
                    hs = pl.ds(h * 16, 16)
                    acc = w[0] * rows[4 * i + 0, p_, hs]
                    acc = acc + w[1] * rows[4 * i + 1, p_, hs]
                    acc = acc + w[2] * rows[4 * i + 2, p_, hs]
                    acc = acc + w[3] * rows[4 * i + 3, p_, hs]
                    v = (u0 * rows[12 + 2 * i, p_, hs]
                         + u1 * rows[13 + 2 * i, p_, hs])
                    outbuf[p_, pl.ds(32 * i + h * 16, 16)] = acc * v
            return carry

        lax.fori_loop(0, P, point_body, None)

        pltpu.sync_copy(outbuf, out_hbm.at[pl.ds(base, P)])
        return None

    lax.fori_loop(0, N_CHUNKS, chunk_body, None)


@jax.jit
def _encode(xT, mat_tab, vec_tab):
    mesh = plsc.VectorSubcoreMesh(core_axis_name="c", subcore_axis_name="s",
                                  num_cores=NC, num_subcores=NS)
    run = pl.kernel(
        _body,
        out_type=jax.ShapeDtypeStruct((N_PTS, 3 * ODIM), jnp.float32),
        mesh=mesh,
        compiler_params=pltpu.CompilerParams(
            use_tc_tiling_on_sc=False, needs_layout_passes=False),
        scratch_types=[
            pltpu.VMEM((3, P), jnp.float32),
            pltpu.VMEM((18, P), jnp.int32),
            pltpu.VMEM((P, 32), jnp.float32),
            pltpu.VMEM((18, P, ODIM), jnp.float32),
            pltpu.VMEM((P, 3 * ODIM), jnp.float32),
            pltpu.SemaphoreType.DMA,
        ],
    )
    return run(xT, mat_tab, vec_tab)


def kernel(x, C_mat, C_vec):
    mat_tab = jnp.transpose(C_mat, (0, 2, 3, 1)).reshape(3 * RES * RES, ODIM)
    vec_tab = jnp.transpose(C_vec[:, :, :, 0], (0, 2, 1)).reshape(3 * RES, ODIM)
    xT = x.T
    return _encode(xT, mat_tab, vec_tab)
